# Initial kernel scaffold; baseline (speedup 1.0000x reference)
import jax
import jax.numpy as jnp
from jax.experimental import pallas as pl


def kernel(edge_index, edge_feat, node_feat, g_repr, W_e1, b_e1, W_e2, b_e2, W_n1, b_n1, W_n2, b_n2, W_u1, b_u1, W_u2, b_u2):
    raise NotImplementedError("write your pallas kernel here")


# speedup vs baseline: 3.9402x; 3.9402x over previous
import functools

import jax
import jax.numpy as jnp
from jax import lax
from jax.experimental import pallas as pl
from jax.experimental.pallas import tpu as pltpu
from jax.experimental.pallas import tpu_sc as plsc

N = 10000
E = 320000
D = 128
D_EDGE = 16
D_U = 32

NC = 2
NS = 16
NW = NC * NS
PW = E // NW
C = 80
NCH = PW // C

_mesh = plsc.VectorSubcoreMesh(core_axis_name="c", subcore_axis_name="s")


def _gather_body(psrc, pdst, src, dst, gs, gd,
                 idx_s, idx_d, bs0, bd0, bs1, bd1, ss0, sd0, ss1, sd1):
    wid = lax.axis_index("s") * NC + lax.axis_index("c")
    base = wid * PW
    pltpu.sync_copy(src.at[pl.ds(base, PW)], idx_s)
    pltpu.sync_copy(dst.at[pl.ds(base, PW)], idx_d)

    def issue(k, bs, bd, sa, sb):
        a = pltpu.async_copy(psrc.at[idx_s.at[pl.ds(k * C, C)]], bs, sa)
        b = pltpu.async_copy(pdst.at[idx_d.at[pl.ds(k * C, C)]], bd, sb)
        return a, b

    def wait0():
        pltpu.make_async_copy(psrc.at[idx_s.at[pl.ds(0, C)]], bs0, ss0).wait()
        pltpu.make_async_copy(pdst.at[idx_d.at[pl.ds(0, C)]], bd0, sd0).wait()

    def store(k, bs, bd):
        pltpu.sync_copy(bs, gs.at[pl.ds(base + k * C, C)])
        pltpu.sync_copy(bd, gd.at[pl.ds(base + k * C, C)])

    issue(0, bs0, bd0, ss0, sd0)

    def outer(i, carry):
        k0 = 2 * i
        da, db = issue(k0 + 1, bs1, bd1, ss1, sd1)
        wait0()
        store(k0, bs0, bd0)
        issue(k0 + 2, bs0, bd0, ss0, sd0)
        da.wait()
        db.wait()
        store(k0 + 1, bs1, bd1)
        return carry

    lax.fori_loop(0, (NCH - 1) // 2, outer, 0)
    wait0()
    store(NCH - 1, bs0, bd0)


_gather_call = pl.kernel(
    _gather_body,
    mesh=_mesh,
    out_type=[jax.ShapeDtypeStruct((E, D), jnp.float32),
              jax.ShapeDtypeStruct((E, D), jnp.float32)],
    scratch_types=[
        pltpu.VMEM((PW,), jnp.int32), pltpu.VMEM((PW,), jnp.int32),
        pltpu.VMEM((C, D), jnp.float32), pltpu.VMEM((C, D), jnp.float32),
        pltpu.VMEM((C, D), jnp.float32), pltpu.VMEM((C, D), jnp.float32),
        pltpu.SemaphoreType.DMA, pltpu.SemaphoreType.DMA,
        pltpu.SemaphoreType.DMA, pltpu.SemaphoreType.DMA,
    ],
)


def _scatter_body(eout, dsti, zeros, hpart,
                  b0, b1, i0, i1, hsh, se0, se1, si0, si1):
    cid = lax.axis_index("c")
    sid = lax.axis_index("s")
    wid = sid * NC + cid
    base = wid * PW
    rz = 624
    tail = N - rz * NS

    pltpu.sync_copy(zeros.at[pl.ds(sid * rz, rz)], hsh.at[pl.ds(sid * rz, rz)])

    @pl.when(sid == 0)
    def _():
        pltpu.sync_copy(zeros.at[pl.ds(rz * NS, tail)],
                        hsh.at[pl.ds(rz * NS, tail)])

    plsc.subcore_barrier()

    def issue(k, bb, ib, sa, sb):
        a = pltpu.async_copy(eout.at[pl.ds(base + k * C, C)], bb, sa)
        b = pltpu.async_copy(dsti.at[pl.ds(base + k * C, C)], ib, sb)
        return a, b

    def wait0():
        pltpu.make_async_copy(eout.at[pl.ds(base, C)], b0, se0).wait()
        pltpu.make_async_copy(dsti.at[pl.ds(base, C)], i0, si0).wait()

    def scat(bb, ib):
        pltpu.sync_copy(bb, hsh.at[ib], add=True)

    issue(0, b0, i0, se0, si0)

    def outer(i, carry):
        k0 = 2 * i
        da, db = issue(k0 + 1, b1, i1, se1, si1)
        wait0()
        scat(b0, i0)
        issue(k0 + 2, b0, i0, se0, si0)
        da.wait()
        db.wait()
        scat(b1, i1)
        return carry

    lax.fori_loop(0, (NCH - 1) // 2, outer, 0)
    wait0()
    scat(b0, i0)

    plsc.subcore_barrier()
    pltpu.sync_copy(hsh.at[pl.ds(sid * rz, rz)],
                    hpart.at[cid, pl.ds(sid * rz, rz)])

    @pl.when(sid == 0)
    def _():
        pltpu.sync_copy(hsh.at[pl.ds(rz * NS, tail)],
                        hpart.at[cid, pl.ds(rz * NS, tail)])


_scatter_call = pl.kernel(
    _scatter_body,
    mesh=_mesh,
    out_type=jax.ShapeDtypeStruct((NC, N, D), jnp.float32),
    scratch_types=[
        pltpu.VMEM((C, D), jnp.float32), pltpu.VMEM((C, D), jnp.float32),
        pltpu.VMEM((C,), jnp.int32), pltpu.VMEM((C,), jnp.int32),
        pltpu.VMEM_SHARED((N, D), jnp.float32),
        pltpu.SemaphoreType.DMA, pltpu.SemaphoreType.DMA,
        pltpu.SemaphoreType.DMA, pltpu.SemaphoreType.DMA,
    ],
)


_B1 = 1000
_BE = 2560


def _prep_body(nf, wsrc, wdst, g, weu, be1, wnu, bn1, psrc_o, pdst_o, ce_o, cn_o):
    nfb = nf[...]
    psrc_o[...] = jnp.dot(nfb, wsrc[...], preferred_element_type=jnp.float32)
    pdst_o[...] = jnp.dot(nfb, wdst[...], preferred_element_type=jnp.float32)

    @pl.when(pl.program_id(0) == 0)
    def _():
        gv = g[...]
        ce_o[...] = jnp.dot(gv, weu[...], preferred_element_type=jnp.float32) + be1[...]
        cn_o[...] = jnp.dot(gv, wnu[...], preferred_element_type=jnp.float32) + bn1[...]


def _edge_body(gs, gd, ef, wef, we2, ce, be2, out):
    pre = gs[...] + gd[...] + ce[...]
    pre = pre + jnp.dot(ef[...], wef[...], preferred_element_type=jnp.float32)
    r = jnp.maximum(pre, 0.0)
    out[...] = jnp.dot(r, we2[...], preferred_element_type=jnp.float32) + be2[...]


def _node_body(nf, hp, wnf, wnh, cn, wn2, bn2, nout_o, comb_o):
    h = hp[0] + hp[1]
    pre = (jnp.dot(nf[...], wnf[...], preferred_element_type=jnp.float32)
           + jnp.dot(h, wnh[...], preferred_element_type=jnp.float32)
           + cn[...])
    r = jnp.maximum(pre, 0.0)
    nout = jnp.dot(r, wn2[...], preferred_element_type=jnp.float32) + bn2[...]
    nout_o[...] = nout
    part = jnp.concatenate([jnp.sum(h, axis=0, keepdims=True),
                            jnp.sum(nout, axis=0, keepdims=True)], axis=0)

    @pl.when(pl.program_id(0) == 0)
    def _():
        comb_o[...] = part

    @pl.when(pl.program_id(0) != 0)
    def _():
        comb_o[...] = comb_o[...] + part


def _final_body(comb, g, wun, wue, wug, bu1, wu2, bu2, out):
    ecomb = comb[0:1, :]
    ncomb = comb[1:2, :]
    pre = (jnp.dot(ncomb, wun[...], preferred_element_type=jnp.float32)
           + jnp.dot(ecomb, wue[...], preferred_element_type=jnp.float32)
           + jnp.dot(g[...], wug[...], preferred_element_type=jnp.float32)
           + bu1[...])
    r = jnp.maximum(pre, 0.0)
    out[...] = jnp.dot(r, wu2[...], preferred_element_type=jnp.float32) + bu2[...]


def _const_spec(shape):
    return pl.BlockSpec(shape, lambda i: tuple(0 for _ in shape))


_prep_call = pl.pallas_call(
    _prep_body,
    grid=(N // _B1,),
    in_specs=[
        pl.BlockSpec((_B1, D), lambda i: (i, 0)),
        _const_spec((D, D)), _const_spec((D, D)),
        _const_spec((1, D_U)), _const_spec((D_U, D)), _const_spec((1, D)),
        _const_spec((D_U, D)), _const_spec((1, D)),
    ],
    out_specs=[
        pl.BlockSpec((_B1, D), lambda i: (i, 0)),
        pl.BlockSpec((_B1, D), lambda i: (i, 0)),
        _const_spec((1, D)), _const_spec((1, D)),
    ],
    out_shape=[
        jax.ShapeDtypeStruct((N, D), jnp.float32),
        jax.ShapeDtypeStruct((N, D), jnp.float32),
        jax.ShapeDtypeStruct((1, D), jnp.float32),
        jax.ShapeDtypeStruct((1, D), jnp.float32),
    ],
)

_edge_call = pl.pallas_call(
    _edge_body,
    grid=(E // _BE,),
    in_specs=[
        pl.BlockSpec((_BE, D), lambda i: (i, 0)),
        pl.BlockSpec((_BE, D), lambda i: (i, 0)),
        pl.BlockSpec((_BE, D_EDGE), lambda i: (i, 0)),
        _const_spec((D_EDGE, D)), _const_spec((D, D)),
        _const_spec((1, D)), _const_spec((1, D)),
    ],
    out_specs=pl.BlockSpec((_BE, D), lambda i: (i, 0)),
    out_shape=jax.ShapeDtypeStruct((E, D), jnp.float32),
)

_node_call = pl.pallas_call(
    _node_body,
    grid=(N // _B1,),
    in_specs=[
        pl.BlockSpec((_B1, D), lambda i: (i, 0)),
        pl.BlockSpec((NC, _B1, D), lambda i: (0, i, 0)),
        _const_spec((D, D)), _const_spec((D, D)), _const_spec((1, D)),
        _const_spec((D, D)), _const_spec((1, D)),
    ],
    out_specs=[
        pl.BlockSpec((_B1, D), lambda i: (i, 0)),
        _const_spec((2, D)),
    ],
    out_shape=[
        jax.ShapeDtypeStruct((N, D), jnp.float32),
        jax.ShapeDtypeStruct((2, D), jnp.float32),
    ],
)

_final_call = pl.pallas_call(
    _final_body,
    out_shape=jax.ShapeDtypeStruct((1, D), jnp.float32),
)


def kernel(edge_index, edge_feat, node_feat, g_repr,
           W_e1, b_e1, W_e2, b_e2, W_n1, b_n1, W_n2, b_n2,
           W_u1, b_u1, W_u2, b_u2):
    src = edge_index[0]
    dst = edge_index[1]

    W_ef = W_e1[:D_EDGE]
    W_es = W_e1[D_EDGE:D_EDGE + D]
    W_ed = W_e1[D_EDGE + D:D_EDGE + 2 * D]
    W_eu = W_e1[D_EDGE + 2 * D:]
    W_nf = W_n1[:D]
    W_nh = W_n1[D:2 * D]
    W_nu = W_n1[2 * D:]
    W_un = W_u1[:D]
    W_ue = W_u1[D:2 * D]
    W_ug = W_u1[2 * D:]

    psrc, pdst, c_e, c_n = _prep_call(
        node_feat, W_es, W_ed, g_repr, W_eu, b_e1.reshape(1, D),
        W_nu, b_n1.reshape(1, D))
    gs, gd = _gather_call(psrc, pdst, src, dst)
    e_out = _edge_call(gs, gd, edge_feat, W_ef, W_e2, c_e,
                       b_e2.reshape(1, D))
    hpart = _scatter_call(e_out, dst, jnp.zeros((N, D), jnp.float32))
    n_out, comb = _node_call(node_feat, hpart, W_nf, W_nh, c_n, W_n2,
                             b_n2.reshape(1, D))
    u_out = _final_call(comb, g_repr, W_un, W_ue, W_ug,
                        b_u1.reshape(1, D), W_u2, b_u2.reshape(1, D))
    return (e_out, n_out, u_out)


# speedup vs baseline: 3.9888x; 1.0123x over previous
import functools

import jax
import jax.numpy as jnp
from jax import lax
from jax.experimental import pallas as pl
from jax.experimental.pallas import tpu as pltpu
from jax.experimental.pallas import tpu_sc as plsc

N = 10000
E = 320000
D = 128
D_EDGE = 16
D_U = 32

NC = 2
NS = 16
NW = NC * NS
C = 80

EA = 163840
EB = E - EA

_mesh = plsc.VectorSubcoreMesh(core_axis_name="c", subcore_axis_name="s")


def _make_gather(e_part):
    pw = e_part // NW
    nch = pw // C

    def body(psrc, pdst, src, dst, out, idx_s, idx_d, b0, b1,
             sg0, sg1, st0, st1):
        wid = lax.axis_index("s") * NC + lax.axis_index("c")
        base = wid * pw
        cbase = wid * nch

        pltpu.sync_copy(src.at[pl.ds(base, pw)], idx_s)
        pltpu.sync_copy(dst.at[pl.ds(base, pw)], idx_d)

        def issue_g(k, bb, sg):
            pltpu.async_copy(psrc.at[idx_s.at[pl.ds(k * C, C)]], bb.at[0], sg)
            pltpu.async_copy(pdst.at[idx_d.at[pl.ds(k * C, C)]], bb.at[1], sg)

        def wait_g(bb, sg):
            pltpu.make_async_copy(
                psrc.at[idx_s.at[pl.ds(0, C)]], bb.at[0], sg).wait()
            pltpu.make_async_copy(
                pdst.at[idx_d.at[pl.ds(0, C)]], bb.at[1], sg).wait()

        def issue_st(k, bb, st):
            pltpu.async_copy(bb, out.at[cbase + k], st)

        def wait_st(bb, st):
            pltpu.make_async_copy(bb, out.at[cbase], st).wait()

        issue_g(0, b0, sg0)
        issue_g(1, b1, sg1)

        def outer(i, carry):
            k0 = 2 * i
            wait_g(b0, sg0)
            issue_st(k0, b0, st0)
            wait_g(b1, sg1)
            issue_st(k0 + 1, b1, st1)
            wait_st(b0, st0)
            issue_g(k0 + 2, b0, sg0)
            wait_st(b1, st1)
            issue_g(k0 + 3, b1, sg1)
            return carry

        if nch % 2 == 0:
            lax.fori_loop(0, (nch - 2) // 2, outer, 0)
            wait_g(b0, sg0)
            issue_st(nch - 2, b0, st0)
            wait_g(b1, sg1)
            issue_st(nch - 1, b1, st1)
            wait_st(b0, st0)
            wait_st(b1, st1)
        else:
            lax.fori_loop(0, (nch - 3) // 2, outer, 0)
            wait_g(b0, sg0)
            issue_st(nch - 3, b0, st0)
            wait_g(b1, sg1)
            issue_st(nch - 2, b1, st1)
            wait_st(b0, st0)
            issue_g(nch - 1, b0, sg0)
            wait_g(b0, sg0)
            issue_st(nch - 1, b0, st0)
            wait_st(b1, st1)
            wait_st(b0, st0)

    return pl.kernel(
        body,
        mesh=_mesh,
        out_type=jax.ShapeDtypeStruct((NW * nch, 2, C, D), jnp.float32),
        scratch_types=[
            pltpu.VMEM((pw,), jnp.int32), pltpu.VMEM((pw,), jnp.int32),
            pltpu.VMEM((2, C, D), jnp.float32),
            pltpu.VMEM((2, C, D), jnp.float32),
            pltpu.SemaphoreType.DMA, pltpu.SemaphoreType.DMA,
            pltpu.SemaphoreType.DMA, pltpu.SemaphoreType.DMA,
        ],
    )


_gather_a = _make_gather(EA)
_gather_b = _make_gather(EB)


PW = E // NW
NCH = PW // C


def _scatter_body(eout, dsti, zeros, hpart,
                  b0, b1, i0, i1, hsh, se0, se1, si0, si1):
    cid = lax.axis_index("c")
    sid = lax.axis_index("s")
    wid = sid * NC + cid
    base = wid * PW
    rz = 624
    tail = N - rz * NS

    pltpu.sync_copy(zeros.at[pl.ds(sid * rz, rz)], hsh.at[pl.ds(sid * rz, rz)])

    @pl.when(sid == 0)
    def _():
        pltpu.sync_copy(zeros.at[pl.ds(rz * NS, tail)],
                        hsh.at[pl.ds(rz * NS, tail)])

    plsc.subcore_barrier()

    def issue(k, bb, ib, sa, sb):
        a = pltpu.async_copy(eout.at[pl.ds(base + k * C, C)], bb, sa)
        b = pltpu.async_copy(dsti.at[pl.ds(base + k * C, C)], ib, sb)
        return a, b

    def wait0():
        pltpu.make_async_copy(eout.at[pl.ds(base, C)], b0, se0).wait()
        pltpu.make_async_copy(dsti.at[pl.ds(base, C)], i0, si0).wait()

    def scat(bb, ib):
        pltpu.sync_copy(bb, hsh.at[ib], add=True)

    issue(0, b0, i0, se0, si0)

    def outer(i, carry):
        k0 = 2 * i
        da, db = issue(k0 + 1, b1, i1, se1, si1)
        wait0()
        scat(b0, i0)
        issue(k0 + 2, b0, i0, se0, si0)
        da.wait()
        db.wait()
        scat(b1, i1)
        return carry

    lax.fori_loop(0, (NCH - 1) // 2, outer, 0)
    wait0()
    scat(b0, i0)

    plsc.subcore_barrier()
    pltpu.sync_copy(hsh.at[pl.ds(sid * rz, rz)],
                    hpart.at[cid, pl.ds(sid * rz, rz)])

    @pl.when(sid == 0)
    def _():
        pltpu.sync_copy(hsh.at[pl.ds(rz * NS, tail)],
                        hpart.at[cid, pl.ds(rz * NS, tail)])


_scatter_call = pl.kernel(
    _scatter_body,
    mesh=_mesh,
    out_type=jax.ShapeDtypeStruct((NC, N, D), jnp.float32),
    scratch_types=[
        pltpu.VMEM((C, D), jnp.float32), pltpu.VMEM((C, D), jnp.float32),
        pltpu.VMEM((C,), jnp.int32), pltpu.VMEM((C,), jnp.int32),
        pltpu.VMEM_SHARED((N, D), jnp.float32),
        pltpu.SemaphoreType.DMA, pltpu.SemaphoreType.DMA,
        pltpu.SemaphoreType.DMA, pltpu.SemaphoreType.DMA,
    ],
)


_B1 = 1000
_BE = 2560
_NBA = EA // _BE


def _prep_body(nf, wsrc, wdst, g, weu, be1, wnu, bn1, psrc_o, pdst_o, ce_o, cn_o):
    nfb = nf[...]
    psrc_o[...] = jnp.dot(nfb, wsrc[...], preferred_element_type=jnp.float32)
    pdst_o[...] = jnp.dot(nfb, wdst[...], preferred_element_type=jnp.float32)

    @pl.when(pl.program_id(0) == 0)
    def _():
        gv = g[...]
        ce_o[...] = jnp.dot(gv, weu[...], preferred_element_type=jnp.float32) + be1[...]
        cn_o[...] = jnp.dot(gv, wnu[...], preferred_element_type=jnp.float32) + bn1[...]


def _edge_compute(gsgd, ef, wef, we2, ce, be2):
    gs = jnp.reshape(gsgd[:, 0, :, :], (_BE, D))
    gd = jnp.reshape(gsgd[:, 1, :, :], (_BE, D))
    pre = gs + gd + ce[...]
    pre = pre + jnp.dot(ef[...], wef[...], preferred_element_type=jnp.float32)
    r = jnp.maximum(pre, 0.0)
    return jnp.dot(r, we2[...], preferred_element_type=jnp.float32) + be2[...]


def _edge_body(gsgd, ef, wef, we2, ce, be2, out, ecol_o):
    eo = _edge_compute(gsgd, ef, wef, we2, ce, be2)
    out[...] = eo
    colsum = jnp.sum(eo, axis=0, keepdims=True)

    @pl.when(pl.program_id(0) == 0)
    def _():
        ecol_o[...] = colsum

    @pl.when(pl.program_id(0) != 0)
    def _():
        ecol_o[...] = ecol_o[...] + colsum


def _edge_body_b(gsgd, ef, prev, eca, wef, we2, ce, be2, out, ecol_o):
    del prev
    eo = _edge_compute(gsgd, ef, wef, we2, ce, be2)
    out[...] = eo
    colsum = jnp.sum(eo, axis=0, keepdims=True)

    @pl.when(pl.program_id(0) == 0)
    def _():
        ecol_o[...] = eca[...] + colsum

    @pl.when(pl.program_id(0) != 0)
    def _():
        ecol_o[...] = ecol_o[...] + colsum


def _node_body(nf, hp, g, ec, wnf, wnh, cn, wn2, bn2, wun, wue, wug, bu1,
               wu2, bu2, nout_o, uout_o, comb):
    h = hp[0] + hp[1]
    pre = (jnp.dot(nf[...], wnf[...], preferred_element_type=jnp.float32)
           + jnp.dot(h, wnh[...], preferred_element_type=jnp.float32)
           + cn[...])
    r = jnp.maximum(pre, 0.0)
    nout = jnp.dot(r, wn2[...], preferred_element_type=jnp.float32) + bn2[...]
    nout_o[...] = nout
    ncol = jnp.sum(nout, axis=0, keepdims=True)

    @pl.when(pl.program_id(0) == 0)
    def _():
        comb[...] = ncol

    @pl.when(pl.program_id(0) != 0)
    def _():
        comb[...] = comb[...] + ncol

    @pl.when(pl.program_id(0) == pl.num_programs(0) - 1)
    def _():
        upre = (jnp.dot(comb[...], wun[...], preferred_element_type=jnp.float32)
                + jnp.dot(ec[...], wue[...], preferred_element_type=jnp.float32)
                + jnp.dot(g[...], wug[...], preferred_element_type=jnp.float32)
                + bu1[...])
        ur = jnp.maximum(upre, 0.0)
        uout_o[...] = jnp.dot(ur, wu2[...], preferred_element_type=jnp.float32) + bu2[...]


def _const_spec(shape):
    return pl.BlockSpec(shape, lambda i: tuple(0 for _ in shape))


_prep_call = pl.pallas_call(
    _prep_body,
    grid=(N // _B1,),
    in_specs=[
        pl.BlockSpec((_B1, D), lambda i: (i, 0)),
        _const_spec((D, D)), _const_spec((D, D)),
        _const_spec((1, D_U)), _const_spec((D_U, D)), _const_spec((1, D)),
        _const_spec((D_U, D)), _const_spec((1, D)),
    ],
    out_specs=[
        pl.BlockSpec((_B1, D), lambda i: (i, 0)),
        pl.BlockSpec((_B1, D), lambda i: (i, 0)),
        _const_spec((1, D)), _const_spec((1, D)),
    ],
    out_shape=[
        jax.ShapeDtypeStruct((N, D), jnp.float32),
        jax.ShapeDtypeStruct((N, D), jnp.float32),
        jax.ShapeDtypeStruct((1, D), jnp.float32),
        jax.ShapeDtypeStruct((1, D), jnp.float32),
    ],
)

_CPB = _BE // C

_edge_call_a = pl.pallas_call(
    _edge_body,
    grid=(EA // _BE,),
    in_specs=[
        pl.BlockSpec((_CPB, 2, C, D), lambda i: (i, 0, 0, 0)),
        pl.BlockSpec((_BE, D_EDGE), lambda i: (i, 0)),
        _const_spec((D_EDGE, D)), _const_spec((D, D)),
        _const_spec((1, D)), _const_spec((1, D)),
    ],
    out_specs=[pl.BlockSpec((_BE, D), lambda i: (i, 0)),
               _const_spec((1, D))],
    out_shape=[jax.ShapeDtypeStruct((E, D), jnp.float32),
               jax.ShapeDtypeStruct((1, D), jnp.float32)],
)

_edge_call_b = pl.pallas_call(
    _edge_body_b,
    grid=(EB // _BE,),
    in_specs=[
        pl.BlockSpec((_CPB, 2, C, D), lambda i: (i, 0, 0, 0)),
        pl.BlockSpec((_BE, D_EDGE), lambda i: (i + EA // _BE, 0)),
        pl.BlockSpec((8, D), lambda i: (0, 0)),
        _const_spec((1, D)),
        _const_spec((D_EDGE, D)), _const_spec((D, D)),
        _const_spec((1, D)), _const_spec((1, D)),
    ],
    out_specs=[pl.BlockSpec((_BE, D), lambda i: (i + EA // _BE, 0)),
               _const_spec((1, D))],
    out_shape=[jax.ShapeDtypeStruct((E, D), jnp.float32),
               jax.ShapeDtypeStruct((1, D), jnp.float32)],
    input_output_aliases={2: 0},
)

_node_call = pl.pallas_call(
    _node_body,
    grid=(N // _B1,),
    in_specs=[
        pl.BlockSpec((_B1, D), lambda i: (i, 0)),
        pl.BlockSpec((NC, _B1, D), lambda i: (0, i, 0)),
        _const_spec((1, D_U)), _const_spec((1, D)),
        _const_spec((D, D)), _const_spec((D, D)), _const_spec((1, D)),
        _const_spec((D, D)), _const_spec((1, D)),
        _const_spec((D, D)), _const_spec((D, D)), _const_spec((D_U, D)),
        _const_spec((1, D)), _const_spec((D, D)), _const_spec((1, D)),
    ],
    out_specs=[
        pl.BlockSpec((_B1, D), lambda i: (i, 0)),
        _const_spec((1, D)),
    ],
    out_shape=[
        jax.ShapeDtypeStruct((N, D), jnp.float32),
        jax.ShapeDtypeStruct((1, D), jnp.float32),
    ],
    scratch_shapes=[pltpu.VMEM((1, D), jnp.float32)],
)


def kernel(edge_index, edge_feat, node_feat, g_repr,
           W_e1, b_e1, W_e2, b_e2, W_n1, b_n1, W_n2, b_n2,
           W_u1, b_u1, W_u2, b_u2):
    src = edge_index[0]
    dst = edge_index[1]

    W_ef = W_e1[:D_EDGE]
    W_es = W_e1[D_EDGE:D_EDGE + D]
    W_ed = W_e1[D_EDGE + D:D_EDGE + 2 * D]
    W_eu = W_e1[D_EDGE + 2 * D:]
    W_nf = W_n1[:D]
    W_nh = W_n1[D:2 * D]
    W_nu = W_n1[2 * D:]
    W_un = W_u1[:D]
    W_ue = W_u1[D:2 * D]
    W_ug = W_u1[2 * D:]

    psrc, pdst, c_e, c_n = _prep_call(
        node_feat, W_es, W_ed, g_repr, W_eu, b_e1.reshape(1, D),
        W_nu, b_n1.reshape(1, D))

    ga = _gather_a(psrc, pdst, src[:EA], dst[:EA])
    gb = _gather_b(psrc, pdst, src[EA:], dst[EA:])

    be2 = b_e2.reshape(1, D)
    e_out0, ecol0 = _edge_call_a(ga, edge_feat, W_ef, W_e2, c_e, be2)
    e_out, ecol = _edge_call_b(gb, edge_feat, e_out0, ecol0, W_ef, W_e2,
                               c_e, be2)

    hpart = _scatter_call(e_out, dst, jnp.zeros((N, D), jnp.float32))

    n_out, u_out = _node_call(
        node_feat, hpart, g_repr, ecol, W_nf, W_nh, c_n, W_n2,
        b_n2.reshape(1, D), W_un, W_ue, W_ug, b_u1.reshape(1, D), W_u2,
        b_u2.reshape(1, D))
    return (e_out, n_out, u_out)
